# baseline (speedup 1.0000x reference)
import functools

import jax
import jax.numpy as jnp
from jax import lax
from jax.experimental import pallas as pl
from jax.experimental.pallas import tpu as pltpu
from jax.experimental.pallas import tpu_sc as plsc

_ROWS = 128
_COLS = 100000
_BLOCK = 2048

_MAIN_STEPS = 35
_SC_START = _MAIN_STEPS * _BLOCK
_SC_COLS = 28672
_TAIL_STEPS = _SC_COLS // _BLOCK

_NWORKERS = 32
_CPW = _SC_COLS // _NWORKERS
_NB = 64
_NBLKS = _CPW // _NB

_K1 = 0
_K2 = 42


def _threefry2x32(k1, k2, x0, x1):
    def rotl(x, d):
        return (x << jnp.uint32(d)) | (x >> jnp.uint32(32 - d))

    def rounds(v0, v1, rots):
        for r in rots:
            v0 = v0 + v1
            v1 = rotl(v1, r)
            v1 = v0 ^ v1
        return v0, v1

    rot_a = (13, 15, 26, 6)
    rot_b = (17, 29, 16, 24)
    ks0 = k1
    ks1 = k2
    ks2 = k1 ^ k2 ^ jnp.uint32(0x1BD11BDA)
    v0 = x0 + ks0
    v1 = x1 + ks1
    v0, v1 = rounds(v0, v1, rot_a)
    v0 = v0 + ks1
    v1 = v1 + (ks2 + jnp.uint32(1))
    v0, v1 = rounds(v0, v1, rot_b)
    v0 = v0 + ks2
    v1 = v1 + (ks0 + jnp.uint32(2))
    v0, v1 = rounds(v0, v1, rot_a)
    v0 = v0 + ks0
    v1 = v1 + (ks1 + jnp.uint32(3))
    v0, v1 = rounds(v0, v1, rot_b)
    v0 = v0 + ks1
    v1 = v1 + (ks2 + jnp.uint32(4))
    v0, v1 = rounds(v0, v1, rot_a)
    v0 = v0 + ks2
    v1 = v1 + (ks0 + jnp.uint32(5))
    return v0, v1


def _bits_to_gumbel(bits):
    fb = (bits >> jnp.uint32(9)) | jnp.uint32(0x3F800000)
    u = lax.bitcast_convert_type(fb, jnp.float32) - jnp.float32(1.0)
    u = u * jnp.float32(1.0 - 1e-20) + jnp.float32(1e-20)
    u = jnp.maximum(jnp.float32(1e-20), u)
    return -jnp.log(-jnp.log(u))


def _main_kernel(lt_ref, vmax_ref, vidx_ref):
    step = pl.program_id(0)

    col = lax.broadcasted_iota(jnp.int32, (_ROWS, _BLOCK), 1) + step * _BLOCK
    row = lax.broadcasted_iota(jnp.int32, (_ROWS, _BLOCK), 0)
    lo = (row * _COLS + col).astype(jnp.uint32)

    o0, o1 = _threefry2x32(
        jnp.uint32(_K1), jnp.uint32(_K2), jnp.uint32(0), lo
    )
    gumbel = _bits_to_gumbel(o0 ^ o1)
    val = lt_ref[...].T + gumbel

    m = jnp.max(val, axis=1, keepdims=True)
    idx = jnp.min(
        jnp.where(val == m, col, jnp.int32(2**31 - 1)), axis=1, keepdims=True
    )

    @pl.when(step == 0)
    def _init():
        vmax_ref[...] = m
        vidx_ref[...] = idx

    @pl.when(step > 0)
    def _update():
        upd = m > vmax_ref[...]
        vmax_ref[...] = jnp.where(upd, m, vmax_ref[...])
        vidx_ref[...] = jnp.where(upd, idx, vidx_ref[...])


def _tail_kernel(
    vmax0_ref, vidx0_ref, bits_ref, lt_ref, vmax_ref, vidx_ref, samp_ref
):
    step = pl.program_id(0)

    @pl.when(step == 0)
    def _init():
        vmax_ref[...] = vmax0_ref[...]
        vidx_ref[...] = vidx0_ref[...]

    col = (
        lax.broadcasted_iota(jnp.int32, (_ROWS, _BLOCK), 1)
        + (_SC_START + step * _BLOCK)
    )
    gumbel = _bits_to_gumbel(bits_ref[...].T)
    val = lt_ref[...].T + gumbel
    val = jnp.where(col < _COLS, val, -jnp.inf)

    m = jnp.max(val, axis=1, keepdims=True)
    idx = jnp.min(
        jnp.where(val == m, col, jnp.int32(2**31 - 1)), axis=1, keepdims=True
    )
    upd = m > vmax_ref[...]
    new_idx = jnp.where(upd, idx, vidx_ref[...])
    vmax_ref[...] = jnp.where(upd, m, vmax_ref[...])
    vidx_ref[...] = new_idx

    @pl.when(step == _TAIL_STEPS - 1)
    def _emit():
        samp_ref[...] = new_idx.T


def _sc_hash_kernel(out_hbm, buf):
    cid = lax.axis_index("c")
    sid = lax.axis_index("s")
    wid = cid * 16 + sid
    base_col = _SC_START + wid * _CPW
    row_off = wid * _CPW
    rowmul = (jnp.arange(16, dtype=jnp.int32) * _COLS).astype(jnp.uint32)

    def blk_body(b, carry):
        def col_body(j, carry2):
            c = base_col + b * _NB + j

            def rg_body(rg, carry3):
                scal = (c + rg * (16 * _COLS)).astype(jnp.uint32)
                o0, o1 = _threefry2x32(
                    jnp.uint32(_K1),
                    jnp.uint32(_K2),
                    jnp.uint32(0),
                    rowmul + scal,
                )
                buf[j, pl.ds(rg * 16, 16)] = o0 ^ o1
                return carry3

            lax.fori_loop(0, 8, rg_body, jnp.int32(0))
            return carry2

        lax.fori_loop(0, _NB, col_body, jnp.int32(0))
        pltpu.sync_copy(buf, out_hbm.at[pl.ds(row_off + b * _NB, _NB)])
        return carry

    lax.fori_loop(0, _NBLKS, blk_body, jnp.int32(0))


@jax.jit
def kernel(logits):
    lt = logits.T

    sc_hash = pl.kernel(
        _sc_hash_kernel,
        out_type=jax.ShapeDtypeStruct((_SC_COLS, _ROWS), jnp.uint32),
        mesh=plsc.VectorSubcoreMesh(core_axis_name="c", subcore_axis_name="s"),
        scratch_types=[pltpu.VMEM((_NB, _ROWS), jnp.uint32)],
    )
    bits = sc_hash()

    pmax, pidx = pl.pallas_call(
        _main_kernel,
        grid=(_MAIN_STEPS,),
        in_specs=[pl.BlockSpec((_BLOCK, _ROWS), lambda i: (i, 0))],
        out_specs=[
            pl.BlockSpec((_ROWS, 1), lambda i: (0, 0)),
            pl.BlockSpec((_ROWS, 1), lambda i: (0, 0)),
        ],
        out_shape=[
            jax.ShapeDtypeStruct((_ROWS, 1), jnp.float32),
            jax.ShapeDtypeStruct((_ROWS, 1), jnp.int32),
        ],
        compiler_params=pltpu.CompilerParams(
            dimension_semantics=("arbitrary",),
        ),
    )(lt)

    _, _, samples = pl.pallas_call(
        _tail_kernel,
        grid=(_TAIL_STEPS,),
        in_specs=[
            pl.BlockSpec((_ROWS, 1), lambda i: (0, 0)),
            pl.BlockSpec((_ROWS, 1), lambda i: (0, 0)),
            pl.BlockSpec((_BLOCK, _ROWS), lambda i: (i, 0)),
            pl.BlockSpec((_BLOCK, _ROWS), lambda i: (i + _MAIN_STEPS, 0)),
        ],
        out_specs=[
            pl.BlockSpec((_ROWS, 1), lambda i: (0, 0)),
            pl.BlockSpec((_ROWS, 1), lambda i: (0, 0)),
            pl.BlockSpec((1, _ROWS), lambda i: (0, 0)),
        ],
        out_shape=[
            jax.ShapeDtypeStruct((_ROWS, 1), jnp.float32),
            jax.ShapeDtypeStruct((_ROWS, 1), jnp.int32),
            jax.ShapeDtypeStruct((1, _ROWS), jnp.int32),
        ],
        compiler_params=pltpu.CompilerParams(
            dimension_semantics=("arbitrary",),
        ),
    )(pmax, pidx, bits, lt)

    return samples.reshape(_ROWS)


# speedup vs baseline: 1.0530x; 1.0530x over previous
import functools

import jax
import jax.numpy as jnp
from jax import lax
from jax.experimental import pallas as pl
from jax.experimental.pallas import tpu as pltpu
from jax.experimental.pallas import tpu_sc as plsc

_ROWS = 128
_COLS = 100000
_BLOCK = 2048

_MAIN_STEPS = 35
_SC_START = _MAIN_STEPS * _BLOCK
_SC_COLS = 28672
_TAIL_STEPS = _SC_COLS // _BLOCK

_NWORKERS = 32
_CPW = _SC_COLS // _NWORKERS
_NB = 64
_NBLKS = _CPW // _NB

_K1 = 0
_K2 = 42


def _threefry2x32(k1, k2, x0, x1):
    def rotl(x, d):
        return (x << jnp.uint32(d)) | (x >> jnp.uint32(32 - d))

    def rounds(v0, v1, rots):
        for r in rots:
            v0 = v0 + v1
            v1 = rotl(v1, r)
            v1 = v0 ^ v1
        return v0, v1

    rot_a = (13, 15, 26, 6)
    rot_b = (17, 29, 16, 24)
    ks0 = k1
    ks1 = k2
    ks2 = k1 ^ k2 ^ jnp.uint32(0x1BD11BDA)
    v0 = x0 + ks0
    v1 = x1 + ks1
    v0, v1 = rounds(v0, v1, rot_a)
    v0 = v0 + ks1
    v1 = v1 + (ks2 + jnp.uint32(1))
    v0, v1 = rounds(v0, v1, rot_b)
    v0 = v0 + ks2
    v1 = v1 + (ks0 + jnp.uint32(2))
    v0, v1 = rounds(v0, v1, rot_a)
    v0 = v0 + ks0
    v1 = v1 + (ks1 + jnp.uint32(3))
    v0, v1 = rounds(v0, v1, rot_b)
    v0 = v0 + ks1
    v1 = v1 + (ks2 + jnp.uint32(4))
    v0, v1 = rounds(v0, v1, rot_a)
    v0 = v0 + ks2
    v1 = v1 + (ks0 + jnp.uint32(5))
    return v0, v1


def _bits_to_gumbel(bits):
    fb = (bits >> jnp.uint32(9)) | jnp.uint32(0x3F800000)
    u = lax.bitcast_convert_type(fb, jnp.float32) - jnp.float32(1.0)
    u = u * jnp.float32(1.0 - 1e-20) + jnp.float32(1e-20)
    u = jnp.maximum(jnp.float32(1e-20), u)
    return -jnp.log(-jnp.log(u))


def _main_kernel(lt_ref, vmax_ref, vidx_ref):
    step = pl.program_id(0)

    col = lax.broadcasted_iota(jnp.int32, (_ROWS, _BLOCK), 1) + step * _BLOCK
    row = lax.broadcasted_iota(jnp.int32, (_ROWS, _BLOCK), 0)
    lo = (row * _COLS + col).astype(jnp.uint32)

    o0, o1 = _threefry2x32(
        jnp.uint32(_K1), jnp.uint32(_K2), jnp.uint32(0), lo
    )
    gumbel = _bits_to_gumbel(o0 ^ o1)
    val = lt_ref[...].T + gumbel

    m = jnp.max(val, axis=1, keepdims=True)
    idx = jnp.min(
        jnp.where(val == m, col, jnp.int32(2**31 - 1)), axis=1, keepdims=True
    )

    @pl.when(step == 0)
    def _init():
        vmax_ref[...] = m
        vidx_ref[...] = idx

    @pl.when(step > 0)
    def _update():
        upd = m > vmax_ref[...]
        vmax_ref[...] = jnp.where(upd, m, vmax_ref[...])
        vidx_ref[...] = jnp.where(upd, idx, vidx_ref[...])


def _tail_kernel(
    vmax0_ref, vidx0_ref, bits_ref, lt_ref, vmax_ref, vidx_ref, samp_ref
):
    step = pl.program_id(0)

    @pl.when(step == 0)
    def _init():
        vmax_ref[...] = vmax0_ref[...]
        vidx_ref[...] = vidx0_ref[...]

    col = (
        lax.broadcasted_iota(jnp.int32, (_ROWS, _BLOCK), 1)
        + (_SC_START + step * _BLOCK)
    )
    gumbel = _bits_to_gumbel(bits_ref[...].T)
    val = lt_ref[...].T + gumbel
    val = jnp.where(col < _COLS, val, -jnp.inf)

    m = jnp.max(val, axis=1, keepdims=True)
    idx = jnp.min(
        jnp.where(val == m, col, jnp.int32(2**31 - 1)), axis=1, keepdims=True
    )
    upd = m > vmax_ref[...]
    new_idx = jnp.where(upd, idx, vidx_ref[...])
    vmax_ref[...] = jnp.where(upd, m, vmax_ref[...])
    vidx_ref[...] = new_idx

    @pl.when(step == _TAIL_STEPS - 1)
    def _emit():
        samp_ref[...] = new_idx.T


def _sc_hash_kernel(out_hbm, buf):
    cid = lax.axis_index("c")
    sid = lax.axis_index("s")
    wid = cid * 16 + sid
    base_col = _SC_START + wid * _CPW
    row_off = wid * _CPW
    rowmul = (jnp.arange(16, dtype=jnp.int32) * _COLS).astype(jnp.uint32)

    def blk_body(b, carry):
        def col_body(j, carry2):
            c = base_col + b * _NB + j
            for rg in range(8):
                scal = (c + rg * (16 * _COLS)).astype(jnp.uint32)
                o0, o1 = _threefry2x32(
                    jnp.uint32(_K1),
                    jnp.uint32(_K2),
                    jnp.uint32(0),
                    rowmul + scal,
                )
                buf[j, pl.ds(rg * 16, 16)] = o0 ^ o1
            return carry2

        lax.fori_loop(0, _NB, col_body, jnp.int32(0))
        pltpu.sync_copy(buf, out_hbm.at[pl.ds(row_off + b * _NB, _NB)])
        return carry

    lax.fori_loop(0, _NBLKS, blk_body, jnp.int32(0))


@jax.jit
def kernel(logits):
    lt = logits.T

    sc_hash = pl.kernel(
        _sc_hash_kernel,
        out_type=jax.ShapeDtypeStruct((_SC_COLS, _ROWS), jnp.uint32),
        mesh=plsc.VectorSubcoreMesh(core_axis_name="c", subcore_axis_name="s"),
        scratch_types=[pltpu.VMEM((_NB, _ROWS), jnp.uint32)],
    )
    bits = sc_hash()

    pmax, pidx = pl.pallas_call(
        _main_kernel,
        grid=(_MAIN_STEPS,),
        in_specs=[pl.BlockSpec((_BLOCK, _ROWS), lambda i: (i, 0))],
        out_specs=[
            pl.BlockSpec((_ROWS, 1), lambda i: (0, 0)),
            pl.BlockSpec((_ROWS, 1), lambda i: (0, 0)),
        ],
        out_shape=[
            jax.ShapeDtypeStruct((_ROWS, 1), jnp.float32),
            jax.ShapeDtypeStruct((_ROWS, 1), jnp.int32),
        ],
        compiler_params=pltpu.CompilerParams(
            dimension_semantics=("arbitrary",),
        ),
    )(lt)

    _, _, samples = pl.pallas_call(
        _tail_kernel,
        grid=(_TAIL_STEPS,),
        in_specs=[
            pl.BlockSpec((_ROWS, 1), lambda i: (0, 0)),
            pl.BlockSpec((_ROWS, 1), lambda i: (0, 0)),
            pl.BlockSpec((_BLOCK, _ROWS), lambda i: (i, 0)),
            pl.BlockSpec((_BLOCK, _ROWS), lambda i: (i + _MAIN_STEPS, 0)),
        ],
        out_specs=[
            pl.BlockSpec((_ROWS, 1), lambda i: (0, 0)),
            pl.BlockSpec((_ROWS, 1), lambda i: (0, 0)),
            pl.BlockSpec((1, _ROWS), lambda i: (0, 0)),
        ],
        out_shape=[
            jax.ShapeDtypeStruct((_ROWS, 1), jnp.float32),
            jax.ShapeDtypeStruct((_ROWS, 1), jnp.int32),
            jax.ShapeDtypeStruct((1, _ROWS), jnp.int32),
        ],
        compiler_params=pltpu.CompilerParams(
            dimension_semantics=("arbitrary",),
        ),
    )(pmax, pidx, bits, lt)

    return samples.reshape(_ROWS)


# speedup vs baseline: 1.0535x; 1.0004x over previous
import functools

import jax
import jax.numpy as jnp
from jax import lax
from jax.experimental import pallas as pl
from jax.experimental.pallas import tpu as pltpu
from jax.experimental.pallas import tpu_sc as plsc

_ROWS = 128
_COLS = 100000
_BLOCK = 2048

_MAIN_STEPS = 35
_SC_START = _MAIN_STEPS * _BLOCK
_SC_COLS = 28672
_TAIL_STEPS = _SC_COLS // _BLOCK

_NWORKERS = 32
_CPW = _SC_COLS // _NWORKERS
_NB = 64
_NBLKS = _CPW // _NB

_K1 = 0
_K2 = 42


def _threefry2x32(k1, k2, x0, x1):
    def rotl(x, d):
        return (x << jnp.uint32(d)) | (x >> jnp.uint32(32 - d))

    def rounds(v0, v1, rots):
        for r in rots:
            v0 = v0 + v1
            v1 = rotl(v1, r)
            v1 = v0 ^ v1
        return v0, v1

    rot_a = (13, 15, 26, 6)
    rot_b = (17, 29, 16, 24)
    ks0 = k1
    ks1 = k2
    ks2 = k1 ^ k2 ^ jnp.uint32(0x1BD11BDA)
    v0 = x0 + ks0
    v1 = x1 + ks1
    v0, v1 = rounds(v0, v1, rot_a)
    v0 = v0 + ks1
    v1 = v1 + (ks2 + jnp.uint32(1))
    v0, v1 = rounds(v0, v1, rot_b)
    v0 = v0 + ks2
    v1 = v1 + (ks0 + jnp.uint32(2))
    v0, v1 = rounds(v0, v1, rot_a)
    v0 = v0 + ks0
    v1 = v1 + (ks1 + jnp.uint32(3))
    v0, v1 = rounds(v0, v1, rot_b)
    v0 = v0 + ks1
    v1 = v1 + (ks2 + jnp.uint32(4))
    v0, v1 = rounds(v0, v1, rot_a)
    v0 = v0 + ks2
    v1 = v1 + (ks0 + jnp.uint32(5))
    return v0, v1


def _bits_to_gumbel(bits):
    fb = (bits >> jnp.uint32(9)) | jnp.uint32(0x3F800000)
    u = lax.bitcast_convert_type(fb, jnp.float32) - jnp.float32(1.0)
    u = u + jnp.float32(1e-20)
    u = jnp.maximum(jnp.float32(1e-20), u)
    return -jnp.log(-jnp.log(u))


def _main_kernel(lt_ref, vmax_ref, vidx_ref):
    step = pl.program_id(0)

    col = lax.broadcasted_iota(jnp.int32, (_ROWS, _BLOCK), 1) + step * _BLOCK
    row = lax.broadcasted_iota(jnp.int32, (_ROWS, _BLOCK), 0)
    lo = (row * _COLS + col).astype(jnp.uint32)

    o0, o1 = _threefry2x32(
        jnp.uint32(_K1), jnp.uint32(_K2), jnp.uint32(0), lo
    )
    gumbel = _bits_to_gumbel(o0 ^ o1)
    val = lt_ref[...].T + gumbel

    m = jnp.max(val, axis=1, keepdims=True)
    idx = jnp.min(
        jnp.where(val == m, col, jnp.int32(2**31 - 1)), axis=1, keepdims=True
    )

    @pl.when(step == 0)
    def _init():
        vmax_ref[...] = m
        vidx_ref[...] = idx

    @pl.when(step > 0)
    def _update():
        upd = m > vmax_ref[...]
        vmax_ref[...] = jnp.where(upd, m, vmax_ref[...])
        vidx_ref[...] = jnp.where(upd, idx, vidx_ref[...])


def _tail_kernel(
    vmax0_ref, vidx0_ref, bits_ref, lt_ref, vmax_ref, vidx_ref, samp_ref
):
    step = pl.program_id(0)

    @pl.when(step == 0)
    def _init():
        vmax_ref[...] = vmax0_ref[...]
        vidx_ref[...] = vidx0_ref[...]

    col = (
        lax.broadcasted_iota(jnp.int32, (_ROWS, _BLOCK), 1)
        + (_SC_START + step * _BLOCK)
    )
    gumbel = _bits_to_gumbel(bits_ref[...].T)
    val = lt_ref[...].T + gumbel
    val = jnp.where(col < _COLS, val, -jnp.inf)

    m = jnp.max(val, axis=1, keepdims=True)
    idx = jnp.min(
        jnp.where(val == m, col, jnp.int32(2**31 - 1)), axis=1, keepdims=True
    )
    upd = m > vmax_ref[...]
    new_idx = jnp.where(upd, idx, vidx_ref[...])
    vmax_ref[...] = jnp.where(upd, m, vmax_ref[...])
    vidx_ref[...] = new_idx

    @pl.when(step == _TAIL_STEPS - 1)
    def _emit():
        samp_ref[...] = new_idx.T


def _sc_hash_kernel(out_hbm, buf):
    cid = lax.axis_index("c")
    sid = lax.axis_index("s")
    wid = cid * 16 + sid
    base_col = _SC_START + wid * _CPW
    row_off = wid * _CPW
    rowmul = (jnp.arange(16, dtype=jnp.int32) * _COLS).astype(jnp.uint32)

    def blk_body(b, carry):
        def col_body(j, carry2):
            c = base_col + b * _NB + j
            for rg in range(8):
                scal = (c + rg * (16 * _COLS)).astype(jnp.uint32)
                o0, o1 = _threefry2x32(
                    jnp.uint32(_K1),
                    jnp.uint32(_K2),
                    jnp.uint32(0),
                    rowmul + scal,
                )
                buf[j, pl.ds(rg * 16, 16)] = o0 ^ o1
            return carry2

        lax.fori_loop(0, _NB, col_body, jnp.int32(0))
        pltpu.sync_copy(buf, out_hbm.at[pl.ds(row_off + b * _NB, _NB)])
        return carry

    lax.fori_loop(0, _NBLKS, blk_body, jnp.int32(0))


@jax.jit
def kernel(logits):
    lt = logits.T

    sc_hash = pl.kernel(
        _sc_hash_kernel,
        out_type=jax.ShapeDtypeStruct((_SC_COLS, _ROWS), jnp.uint32),
        mesh=plsc.VectorSubcoreMesh(core_axis_name="c", subcore_axis_name="s"),
        scratch_types=[pltpu.VMEM((_NB, _ROWS), jnp.uint32)],
    )
    bits = sc_hash()

    pmax, pidx = pl.pallas_call(
        _main_kernel,
        grid=(_MAIN_STEPS,),
        in_specs=[pl.BlockSpec((_BLOCK, _ROWS), lambda i: (i, 0))],
        out_specs=[
            pl.BlockSpec((_ROWS, 1), lambda i: (0, 0)),
            pl.BlockSpec((_ROWS, 1), lambda i: (0, 0)),
        ],
        out_shape=[
            jax.ShapeDtypeStruct((_ROWS, 1), jnp.float32),
            jax.ShapeDtypeStruct((_ROWS, 1), jnp.int32),
        ],
        compiler_params=pltpu.CompilerParams(
            dimension_semantics=("arbitrary",),
        ),
    )(lt)

    _, _, samples = pl.pallas_call(
        _tail_kernel,
        grid=(_TAIL_STEPS,),
        in_specs=[
            pl.BlockSpec((_ROWS, 1), lambda i: (0, 0)),
            pl.BlockSpec((_ROWS, 1), lambda i: (0, 0)),
            pl.BlockSpec((_BLOCK, _ROWS), lambda i: (i, 0)),
            pl.BlockSpec((_BLOCK, _ROWS), lambda i: (i + _MAIN_STEPS, 0)),
        ],
        out_specs=[
            pl.BlockSpec((_ROWS, 1), lambda i: (0, 0)),
            pl.BlockSpec((_ROWS, 1), lambda i: (0, 0)),
            pl.BlockSpec((1, _ROWS), lambda i: (0, 0)),
        ],
        out_shape=[
            jax.ShapeDtypeStruct((_ROWS, 1), jnp.float32),
            jax.ShapeDtypeStruct((_ROWS, 1), jnp.int32),
            jax.ShapeDtypeStruct((1, _ROWS), jnp.int32),
        ],
        compiler_params=pltpu.CompilerParams(
            dimension_semantics=("arbitrary",),
        ),
    )(pmax, pidx, bits, lt)

    return samples.reshape(_ROWS)
